# baseline (speedup 1.0000x reference)
import jax
import jax.numpy as jnp
from jax import lax
from jax.experimental import pallas as pl
from jax.experimental.pallas import tpu as pltpu
from jax.experimental.pallas import tpu_sc as plsc

_CONTEXT = 2
_D0, _D1 = 96, 32
_D = _D0 + _D1
_NC, _NS = 2, 16
_NW = _NC * _NS
_CHUNK = 128
_EPS = 1e-5
_L = 16


def _rsqrt16(x):
    h = x * 0.5
    i = plsc.bitcast(x, jnp.int32)
    g = plsc.bitcast(jnp.full((_L,), 0x5F3759DF, jnp.int32) - (i >> 1),
                     jnp.float32)
    for _ in range(2):
        g = g * (1.5 - h * g * g)
    return g


def _make_body(U):
    def _sc_body(y_hbm, comb_hbm, out_hbm,
                 i_all, gs, outs, tails, g0, g1, w0, w1):
        wid = lax.axis_index("s") * _NC + lax.axis_index("c")
        per_w = out_hbm.shape[0] // _NW
        n_chunks = per_w // _CHUNK
        base_w = wid * per_w
        gsems = (g0, g1)
        wsems = (w0, w1)

        pltpu.sync_copy(y_hbm.at[pl.ds(base_w, per_w)], i_all)

        def gather(slot, ci):
            sl = pl.ds(ci * _CHUNK, _CHUNK)
            return pltpu.make_async_copy(
                comb_hbm.at[i_all.at[sl]], gs.at[slot], gsems[slot])

        def save_tail(slot):
            tails[slot, pl.ds(0, _L)] = gs[slot, _CHUNK - 1, pl.ds(_D0, _L)]
            tails[slot, pl.ds(_L, _L)] = \
                gs[slot, _CHUNK - 1, pl.ds(_D0 + _L, _L)]

        def out_copy(slot, ci):
            base = base_w + ci * _CHUNK
            return pltpu.make_async_copy(
                outs.at[slot], out_hbm.at[pl.ds(base, _CHUNK)], wsems[slot])

        def compute(slot, ci):
            g = gs.at[slot]
            out_v = outs.at[slot]
            base = base_w + ci * _CHUNK
            t_zero = lax.rem(U - lax.rem(base, U), U)

            @plsc.parallel_loop(0, _CHUNK, unroll=4)
            def _tok(t):
                vs = [g[t, pl.ds(_L * j, _L)] for j in range(_D0 // _L)]
                tp = jnp.maximum(t - 1, 0)
                tv = jnp.full((_L,), t, jnp.int32)
                first = tv == 0
                e1a = jnp.where(first, tails[1 - slot, pl.ds(0, _L)],
                                g[tp, pl.ds(_D0, _L)])
                e1b = jnp.where(first, tails[1 - slot, pl.ds(_L, _L)],
                                g[tp, pl.ds(_D0 + _L, _L)])
                row0 = tv == t_zero
                vs.append(jnp.where(row0, 0.0, e1a))
                vs.append(jnp.where(row0, 0.0, e1b))

                s = ((vs[0] + vs[1]) + (vs[2] + vs[3])) + \
                    ((vs[4] + vs[5]) + (vs[6] + vs[7]))
                q = ((vs[0] * vs[0] + vs[1] * vs[1]) +
                     (vs[2] * vs[2] + vs[3] * vs[3])) + \
                    ((vs[4] * vs[4] + vs[5] * vs[5]) +
                     (vs[6] * vs[6] + vs[7] * vs[7]))
                mean = jnp.sum(s) * (1.0 / _D)
                var = jnp.sum(q) * (1.0 / _D) - mean * mean + _EPS
                r = _rsqrt16(jnp.full((_L,), var, jnp.float32))
                m = jnp.full((_L,), mean, jnp.float32)
                for j in range(_D // _L):
                    out_v[t, pl.ds(_L * j, _L)] = (vs[j] - m) * r

        gather(0, 0).start()

        @pl.loop(0, n_chunks, step=2)
        def _pair(c):
            gather(1, c + 1).start()
            gather(0, c).wait()
            save_tail(0)

            @pl.when(c >= 2)
            def _():
                out_copy(0, c).wait()
            compute(0, c)
            out_copy(0, c).start()

            @pl.when(c + 2 < n_chunks)
            def _():
                gather(0, c + 2).start()
            gather(1, c + 1).wait()
            save_tail(1)

            @pl.when(c >= 2)
            def _():
                out_copy(1, c + 1).wait()
            compute(1, c + 1)
            out_copy(1, c + 1).start()

        out_copy(0, n_chunks - 2).wait()
        out_copy(1, n_chunks - 1).wait()

    return _sc_body


_VPAD = 100096
_NFULL = _VPAD // _CHUNK - 1
_TAILV = 100001 - _NFULL * _CHUNK


def _transpose_body(e0t_hbm, e1t_hbm, tail_hbm, comb_hbm,
                    t0s, t1s, obs, i0a, i1a, i0b, i1b, wa, wb):
    wid = lax.axis_index("s") * _NC + lax.axis_index("c")
    lanes = jnp.arange(_L, dtype=jnp.int32)
    isems = ((i0a, i1a), (i0b, i1b))
    wsems = (wa, wb)

    def in_copies(slot, j):
        s0, s1 = isems[slot]
        sl = pl.ds(j * _CHUNK, _CHUNK)
        return (pltpu.make_async_copy(e0t_hbm.at[:, sl], t0s.at[slot], s0),
                pltpu.make_async_copy(e1t_hbm.at[:, sl], t1s.at[slot], s1))

    def out_copy(slot, j):
        return pltpu.make_async_copy(
            obs.at[slot], comb_hbm.at[pl.ds(j * _CHUNK, _CHUNK)],
            wsems[slot])

    def compute(slot):
        t0 = t0s.at[slot]
        t1 = t1s.at[slot]
        ob = obs.at[slot]

        def tile_t(src, dst_col0, i):
            fb = (i >> 3) * _L
            vb = (i & 7) * _L
            fcol = dst_col0 + fb + lanes
            for k in range(_L):
                rot = vb + ((lanes + k) & (_L - 1))
                d = plsc.load_gather(src, [fb + lanes, rot])
                plsc.store_scatter(ob, [rot, fcol], d)

        @plsc.parallel_loop(0, (_D0 // _L) * 8, unroll=2)
        def _t0tile(i):
            tile_t(t0, 0, i)

        @plsc.parallel_loop(0, (_D1 // _L) * 8, unroll=2)
        def _t1tile(i):
            tile_t(t1, _D0, i)

    def step(slot, j, k):
        @pl.when(j < _NFULL)
        def _():
            a, b = in_copies(slot, j)
            a.wait()
            b.wait()

            @pl.when(k >= 2)
            def _():
                out_copy(slot, j).wait()
            compute(slot)
            out_copy(slot, j).start()

    def fire(slot, j):
        @pl.when(j < _NFULL)
        def _():
            a, b = in_copies(slot, j)
            a.start()
            b.start()

    fire(0, wid)

    @pl.loop(0, 25, step=2)
    def _blk(k):
        j0 = k * _NW + wid
        j1 = j0 + _NW
        fire(1, j1)
        step(0, j0, k)
        fire(0, j1 + _NW)
        step(1, j1, k)

    out_copy(0, wid).wait()
    out_copy(1, wid).wait()

    @pl.when(wid == _NW - 1)
    def _():
        nt = tail_hbm.shape[0]
        pltpu.sync_copy(tail_hbm, obs.at[0, pl.ds(0, nt)])
        pltpu.sync_copy(obs.at[0, pl.ds(0, nt)],
                        comb_hbm.at[pl.ds(_NFULL * _CHUNK, nt)])


def kernel(y, emb0, emb1):
    B, U = y.shape
    n_tok = B * U
    per_w = n_tok // _NW
    y_flat = y.reshape(n_tok)

    cp = pltpu.CompilerParams(
        needs_layout_passes=False, use_tc_tiling_on_sc=True)
    mesh = plsc.VectorSubcoreMesh(core_axis_name="c", subcore_axis_name="s")

    transpose_run = pl.kernel(
        _transpose_body,
        compiler_params=cp,
        out_type=jax.ShapeDtypeStruct((_VPAD, _D), jnp.float32),
        mesh=mesh,
        scratch_types=[
            pltpu.VMEM((2, _D0, _CHUNK), jnp.float32),
            pltpu.VMEM((2, _D1, _CHUNK), jnp.float32),
            pltpu.VMEM((2, _CHUNK, _D), jnp.float32),
            pltpu.SemaphoreType.DMA,
            pltpu.SemaphoreType.DMA,
            pltpu.SemaphoreType.DMA,
            pltpu.SemaphoreType.DMA,
            pltpu.SemaphoreType.DMA,
            pltpu.SemaphoreType.DMA,
        ],
    )
    vtail = _NFULL * _CHUNK
    tail = jnp.concatenate([emb0[vtail:], emb1[vtail:]], axis=1)
    tail = jnp.pad(tail, ((0, 40 - tail.shape[0]), (0, 0)))
    comb = transpose_run(emb0.T, emb1.T, tail)

    run = pl.kernel(
        _make_body(U),
        compiler_params=cp,
        out_type=jax.ShapeDtypeStruct((n_tok, _D), jnp.float32),
        mesh=mesh,
        scratch_types=[
            pltpu.VMEM((per_w,), jnp.int32),
            pltpu.VMEM((2, _CHUNK, _D), jnp.float32),
            pltpu.VMEM((2, _CHUNK, _D), jnp.float32),
            pltpu.VMEM((2, 2 * _L), jnp.float32),
            pltpu.SemaphoreType.DMA,
            pltpu.SemaphoreType.DMA,
            pltpu.SemaphoreType.DMA,
            pltpu.SemaphoreType.DMA,
        ],
    )
    out = run(y_flat, comb).reshape(B, U, _D)
    state = y[:, U - _CONTEXT + 1:]
    return (out, state)


# speedup vs baseline: 1.1881x; 1.1881x over previous
import jax
import jax.numpy as jnp
from jax import lax
from jax.experimental import pallas as pl
from jax.experimental.pallas import tpu as pltpu
from jax.experimental.pallas import tpu_sc as plsc

_CONTEXT = 2
_D0, _D1 = 96, 32
_D = _D0 + _D1
_NC, _NS = 2, 16
_NW = _NC * _NS
_CHUNK = 128
_EPS = 1e-5
_L = 16


def _rsqrt16(x):
    h = x * 0.5
    i = plsc.bitcast(x, jnp.int32)
    g = plsc.bitcast(jnp.full((_L,), 0x5F3759DF, jnp.int32) - (i >> 1),
                     jnp.float32)
    for _ in range(2):
        g = g * (1.5 - h * g * g)
    return g


def _make_body(U):
    def _sc_body(y_hbm, comb_hbm, out_hbm,
                 i_all, gs, outs, tails, g0, g1, w0, w1):
        wid = lax.axis_index("s") * _NC + lax.axis_index("c")
        per_w = out_hbm.shape[0] // _NW
        n_chunks = per_w // _CHUNK
        base_w = wid * per_w
        gsems = (g0, g1)
        wsems = (w0, w1)

        pltpu.sync_copy(y_hbm.at[pl.ds(base_w, per_w)], i_all)

        def gather(slot, ci):
            sl = pl.ds(ci * _CHUNK, _CHUNK)
            return pltpu.make_async_copy(
                comb_hbm.at[i_all.at[sl]], gs.at[slot], gsems[slot])

        def save_tail(slot):
            tails[slot, pl.ds(0, _L)] = gs[slot, _CHUNK - 1, pl.ds(_D0, _L)]
            tails[slot, pl.ds(_L, _L)] = \
                gs[slot, _CHUNK - 1, pl.ds(_D0 + _L, _L)]

        def out_copy(slot, ci):
            base = base_w + ci * _CHUNK
            return pltpu.make_async_copy(
                outs.at[slot], out_hbm.at[pl.ds(base, _CHUNK)], wsems[slot])

        def compute(slot, ci):
            g = gs.at[slot]
            out_v = outs.at[slot]
            base = base_w + ci * _CHUNK

            @plsc.parallel_loop(0, _CHUNK, unroll=4)
            def _tok(t):
                vs = [g[t, pl.ds(_L * j, _L)] for j in range(_D0 // _L)]
                tp = jnp.maximum(t - 1, 0)
                tv = jnp.full((_L,), t, jnp.int32)
                first = tv == 0
                e1a = jnp.where(first, tails[1 - slot, pl.ds(0, _L)],
                                g[tp, pl.ds(_D0, _L)])
                e1b = jnp.where(first, tails[1 - slot, pl.ds(_L, _L)],
                                g[tp, pl.ds(_D0 + _L, _L)])
                rem = lax.rem(base + t, U)
                row0 = jnp.full((_L,), rem, jnp.int32) == 0
                vs.append(jnp.where(row0, 0.0, e1a))
                vs.append(jnp.where(row0, 0.0, e1b))

                s = ((vs[0] + vs[1]) + (vs[2] + vs[3])) + \
                    ((vs[4] + vs[5]) + (vs[6] + vs[7]))
                q = ((vs[0] * vs[0] + vs[1] * vs[1]) +
                     (vs[2] * vs[2] + vs[3] * vs[3])) + \
                    ((vs[4] * vs[4] + vs[5] * vs[5]) +
                     (vs[6] * vs[6] + vs[7] * vs[7]))
                mean = jnp.sum(s) * (1.0 / _D)
                var = jnp.sum(q) * (1.0 / _D) - mean * mean + _EPS
                r = _rsqrt16(jnp.full((_L,), var, jnp.float32))
                m = jnp.full((_L,), mean, jnp.float32)
                for j in range(_D // _L):
                    out_v[t, pl.ds(_L * j, _L)] = (vs[j] - m) * r

        gather(0, 0).start()

        @pl.loop(0, n_chunks, step=2)
        def _pair(c):
            gather(1, c + 1).start()
            gather(0, c).wait()
            save_tail(0)

            @pl.when(c >= 2)
            def _():
                out_copy(0, c).wait()
            compute(0, c)
            out_copy(0, c).start()

            @pl.when(c + 2 < n_chunks)
            def _():
                gather(0, c + 2).start()
            gather(1, c + 1).wait()
            save_tail(1)

            @pl.when(c >= 2)
            def _():
                out_copy(1, c + 1).wait()
            compute(1, c + 1)
            out_copy(1, c + 1).start()

        out_copy(0, n_chunks - 2).wait()
        out_copy(1, n_chunks - 1).wait()

    return _sc_body


_VPAD = 100096
_NFULL = _VPAD // _CHUNK - 1
_TAILV = 100001 - _NFULL * _CHUNK


def _transpose_body(e0t_hbm, e1t_hbm, tail_hbm, comb_hbm,
                    t0s, t1s, obs, i0a, i1a, i0b, i1b, wa, wb):
    wid = lax.axis_index("s") * _NC + lax.axis_index("c")
    lanes = jnp.arange(_L, dtype=jnp.int32)
    isems = ((i0a, i1a), (i0b, i1b))
    wsems = (wa, wb)

    def in_copies(slot, j):
        s0, s1 = isems[slot]
        sl = pl.ds(j * _CHUNK, _CHUNK)
        return (pltpu.make_async_copy(e0t_hbm.at[:, sl], t0s.at[slot], s0),
                pltpu.make_async_copy(e1t_hbm.at[:, sl], t1s.at[slot], s1))

    def out_copy(slot, j):
        return pltpu.make_async_copy(
            obs.at[slot], comb_hbm.at[pl.ds(j * _CHUNK, _CHUNK)],
            wsems[slot])

    def compute(slot):
        t0 = t0s.at[slot]
        t1 = t1s.at[slot]
        ob = obs.at[slot]

        def tile_t(src, dst_col0, i):
            fb = (i >> 3) * _L
            vb = (i & 7) * _L
            fcol = dst_col0 + fb + lanes
            for k in range(_L):
                rot = vb + ((lanes + k) & (_L - 1))
                d = plsc.load_gather(src, [fb + lanes, rot])
                plsc.store_scatter(ob, [rot, fcol], d)

        @plsc.parallel_loop(0, (_D0 // _L) * 8, unroll=2)
        def _t0tile(i):
            tile_t(t0, 0, i)

        @plsc.parallel_loop(0, (_D1 // _L) * 8, unroll=2)
        def _t1tile(i):
            tile_t(t1, _D0, i)

    def step(slot, j, k):
        @pl.when(j < _NFULL)
        def _():
            a, b = in_copies(slot, j)
            a.wait()
            b.wait()

            @pl.when(k >= 2)
            def _():
                out_copy(slot, j).wait()
            compute(slot)
            out_copy(slot, j).start()

    def fire(slot, j):
        @pl.when(j < _NFULL)
        def _():
            a, b = in_copies(slot, j)
            a.start()
            b.start()

    fire(0, wid)

    @pl.loop(0, 25, step=2)
    def _blk(k):
        j0 = k * _NW + wid
        j1 = j0 + _NW
        fire(1, j1)
        step(0, j0, k)
        fire(0, j1 + _NW)
        step(1, j1, k)

    out_copy(0, wid).wait()
    out_copy(1, wid).wait()

    @pl.when(wid == _NW - 1)
    def _():
        nt = tail_hbm.shape[0]
        pltpu.sync_copy(tail_hbm, obs.at[0, pl.ds(0, nt)])
        pltpu.sync_copy(obs.at[0, pl.ds(0, nt)],
                        comb_hbm.at[pl.ds(_NFULL * _CHUNK, nt)])


def kernel(y, emb0, emb1):
    B, U = y.shape
    n_tok = B * U
    per_w = n_tok // _NW
    y_flat = y.reshape(n_tok)

    cp = pltpu.CompilerParams(
        needs_layout_passes=False, use_tc_tiling_on_sc=True)
    mesh = plsc.VectorSubcoreMesh(core_axis_name="c", subcore_axis_name="s")

    transpose_run = pl.kernel(
        _transpose_body,
        compiler_params=cp,
        out_type=jax.ShapeDtypeStruct((_VPAD, _D), jnp.float32),
        mesh=mesh,
        scratch_types=[
            pltpu.VMEM((2, _D0, _CHUNK), jnp.float32),
            pltpu.VMEM((2, _D1, _CHUNK), jnp.float32),
            pltpu.VMEM((2, _CHUNK, _D), jnp.float32),
            pltpu.SemaphoreType.DMA,
            pltpu.SemaphoreType.DMA,
            pltpu.SemaphoreType.DMA,
            pltpu.SemaphoreType.DMA,
            pltpu.SemaphoreType.DMA,
            pltpu.SemaphoreType.DMA,
        ],
    )
    vtail = _NFULL * _CHUNK
    tail = jnp.concatenate([emb0[vtail:], emb1[vtail:]], axis=1)
    tail = jnp.pad(tail, ((0, 40 - tail.shape[0]), (0, 0)))
    comb = transpose_run(emb0.T, emb1.T, tail)

    run = pl.kernel(
        _make_body(U),
        compiler_params=cp,
        out_type=jax.ShapeDtypeStruct((n_tok, _D), jnp.float32),
        mesh=mesh,
        scratch_types=[
            pltpu.VMEM((per_w,), jnp.int32),
            pltpu.VMEM((2, _CHUNK, _D), jnp.float32),
            pltpu.VMEM((2, _CHUNK, _D), jnp.float32),
            pltpu.VMEM((2, 2 * _L), jnp.float32),
            pltpu.SemaphoreType.DMA,
            pltpu.SemaphoreType.DMA,
            pltpu.SemaphoreType.DMA,
            pltpu.SemaphoreType.DMA,
        ],
    )
    out = run(y_flat, comb).reshape(B, U, _D)
    state = y[:, U - _CONTEXT + 1:]
    return (out, state)


# speedup vs baseline: 1.2794x; 1.0768x over previous
import jax
import jax.numpy as jnp
from jax import lax
from jax.experimental import pallas as pl
from jax.experimental.pallas import tpu as pltpu
from jax.experimental.pallas import tpu_sc as plsc

_CONTEXT = 2
_D0, _D1 = 96, 32
_D = _D0 + _D1
_NC, _NS = 2, 16
_NW = _NC * _NS
_CHUNK = 128
_EPS = 1e-5
_L = 16


def _rsqrt16(x):
    h = x * 0.5
    i = plsc.bitcast(x, jnp.int32)
    g = plsc.bitcast(jnp.full((_L,), 0x5F3759DF, jnp.int32) - (i >> 1),
                     jnp.float32)
    for _ in range(2):
        g = g * (1.5 - h * g * g)
    return g


def _make_body(U):
    def _sc_body(y_hbm, comb_hbm, out_hbm,
                 i_all, gs, outs, tails, g0, g1, w0, w1):
        wid = lax.axis_index("s") * _NC + lax.axis_index("c")
        per_w = out_hbm.shape[0] // _NW
        n_chunks = per_w // _CHUNK
        base_w = wid * per_w
        gsems = (g0, g1)
        wsems = (w0, w1)

        pltpu.sync_copy(y_hbm.at[pl.ds(base_w, per_w)], i_all)

        def gather(slot, ci):
            sl = pl.ds(ci * _CHUNK, _CHUNK)
            return pltpu.make_async_copy(
                comb_hbm.at[i_all.at[sl]], gs.at[slot], gsems[slot])

        def save_tail(slot):
            tails[slot, pl.ds(0, _L)] = gs[slot, _CHUNK - 1, pl.ds(_D0, _L)]
            tails[slot, pl.ds(_L, _L)] = \
                gs[slot, _CHUNK - 1, pl.ds(_D0 + _L, _L)]

        def out_copy(slot, ci):
            base = base_w + ci * _CHUNK
            return pltpu.make_async_copy(
                outs.at[slot], out_hbm.at[pl.ds(base, _CHUNK)], wsems[slot])

        def compute(slot, ci):
            g = gs.at[slot]
            out_v = outs.at[slot]
            base = base_w + ci * _CHUNK

            @plsc.parallel_loop(0, _CHUNK, unroll=8)
            def _tok(t):
                vs = [g[t, pl.ds(_L * j, _L)] for j in range(_D0 // _L)]
                tp = jnp.maximum(t - 1, 0)
                tv = jnp.full((_L,), t, jnp.int32)
                first = tv == 0
                e1a = jnp.where(first, tails[1 - slot, pl.ds(0, _L)],
                                g[tp, pl.ds(_D0, _L)])
                e1b = jnp.where(first, tails[1 - slot, pl.ds(_L, _L)],
                                g[tp, pl.ds(_D0 + _L, _L)])
                rem = lax.rem(base + t, U)
                row0 = jnp.full((_L,), rem, jnp.int32) == 0
                vs.append(jnp.where(row0, 0.0, e1a))
                vs.append(jnp.where(row0, 0.0, e1b))

                s = ((vs[0] + vs[1]) + (vs[2] + vs[3])) + \
                    ((vs[4] + vs[5]) + (vs[6] + vs[7]))
                q = ((vs[0] * vs[0] + vs[1] * vs[1]) +
                     (vs[2] * vs[2] + vs[3] * vs[3])) + \
                    ((vs[4] * vs[4] + vs[5] * vs[5]) +
                     (vs[6] * vs[6] + vs[7] * vs[7]))
                mean = jnp.sum(s) * (1.0 / _D)
                var = jnp.sum(q) * (1.0 / _D) - mean * mean + _EPS
                r = _rsqrt16(jnp.full((_L,), var, jnp.float32))
                m = jnp.full((_L,), mean, jnp.float32)
                for j in range(_D // _L):
                    out_v[t, pl.ds(_L * j, _L)] = (vs[j] - m) * r

        gather(0, 0).start()

        @pl.loop(0, n_chunks, step=2)
        def _pair(c):
            gather(1, c + 1).start()
            gather(0, c).wait()
            save_tail(0)

            @pl.when(c >= 2)
            def _():
                out_copy(0, c).wait()
            compute(0, c)
            out_copy(0, c).start()

            @pl.when(c + 2 < n_chunks)
            def _():
                gather(0, c + 2).start()
            gather(1, c + 1).wait()
            save_tail(1)

            @pl.when(c >= 2)
            def _():
                out_copy(1, c + 1).wait()
            compute(1, c + 1)
            out_copy(1, c + 1).start()

        out_copy(0, n_chunks - 2).wait()
        out_copy(1, n_chunks - 1).wait()

    return _sc_body


_VPAD = 100096
_NFULL = _VPAD // _CHUNK - 1
_TAILV = 100001 - _NFULL * _CHUNK


def _transpose_body(e0t_hbm, e1t_hbm, tail_hbm, comb_hbm,
                    t0s, t1s, obs, i0a, i1a, i0b, i1b, wa, wb):
    wid = lax.axis_index("s") * _NC + lax.axis_index("c")
    lanes = jnp.arange(_L, dtype=jnp.int32)
    isems = ((i0a, i1a), (i0b, i1b))
    wsems = (wa, wb)

    def in_copies(slot, j):
        s0, s1 = isems[slot]
        sl = pl.ds(j * _CHUNK, _CHUNK)
        return (pltpu.make_async_copy(e0t_hbm.at[:, sl], t0s.at[slot], s0),
                pltpu.make_async_copy(e1t_hbm.at[:, sl], t1s.at[slot], s1))

    def out_copy(slot, j):
        return pltpu.make_async_copy(
            obs.at[slot], comb_hbm.at[pl.ds(j * _CHUNK, _CHUNK)],
            wsems[slot])

    def compute(slot):
        t0 = t0s.at[slot]
        t1 = t1s.at[slot]
        ob = obs.at[slot]

        def tile_t(src, dst_col0, i):
            fb = (i >> 3) * _L
            vb = (i & 7) * _L
            fcol = dst_col0 + fb + lanes
            for k in range(_L):
                rot = vb + ((lanes + k) & (_L - 1))
                d = plsc.load_gather(src, [fb + lanes, rot])
                plsc.store_scatter(ob, [rot, fcol], d)

        @plsc.parallel_loop(0, (_D0 // _L) * 8, unroll=2)
        def _t0tile(i):
            tile_t(t0, 0, i)

        @plsc.parallel_loop(0, (_D1 // _L) * 8, unroll=2)
        def _t1tile(i):
            tile_t(t1, _D0, i)

    def step(slot, j, k):
        @pl.when(j < _NFULL)
        def _():
            a, b = in_copies(slot, j)
            a.wait()
            b.wait()

            @pl.when(k >= 2)
            def _():
                out_copy(slot, j).wait()
            compute(slot)
            out_copy(slot, j).start()

    def fire(slot, j):
        @pl.when(j < _NFULL)
        def _():
            a, b = in_copies(slot, j)
            a.start()
            b.start()

    fire(0, wid)

    @pl.loop(0, 25, step=2)
    def _blk(k):
        j0 = k * _NW + wid
        j1 = j0 + _NW
        fire(1, j1)
        step(0, j0, k)
        fire(0, j1 + _NW)
        step(1, j1, k)

    out_copy(0, wid).wait()
    out_copy(1, wid).wait()

    @pl.when(wid == _NW - 1)
    def _():
        nt = tail_hbm.shape[0]
        pltpu.sync_copy(tail_hbm, obs.at[0, pl.ds(0, nt)])
        pltpu.sync_copy(obs.at[0, pl.ds(0, nt)],
                        comb_hbm.at[pl.ds(_NFULL * _CHUNK, nt)])


def kernel(y, emb0, emb1):
    B, U = y.shape
    n_tok = B * U
    per_w = n_tok // _NW
    y_flat = y.reshape(n_tok)

    cp = pltpu.CompilerParams(
        needs_layout_passes=False, use_tc_tiling_on_sc=True)
    mesh = plsc.VectorSubcoreMesh(core_axis_name="c", subcore_axis_name="s")

    transpose_run = pl.kernel(
        _transpose_body,
        compiler_params=cp,
        out_type=jax.ShapeDtypeStruct((_VPAD, _D), jnp.float32),
        mesh=mesh,
        scratch_types=[
            pltpu.VMEM((2, _D0, _CHUNK), jnp.float32),
            pltpu.VMEM((2, _D1, _CHUNK), jnp.float32),
            pltpu.VMEM((2, _CHUNK, _D), jnp.float32),
            pltpu.SemaphoreType.DMA,
            pltpu.SemaphoreType.DMA,
            pltpu.SemaphoreType.DMA,
            pltpu.SemaphoreType.DMA,
            pltpu.SemaphoreType.DMA,
            pltpu.SemaphoreType.DMA,
        ],
    )
    vtail = _NFULL * _CHUNK
    tail = jnp.concatenate([emb0[vtail:], emb1[vtail:]], axis=1)
    tail = jnp.pad(tail, ((0, 40 - tail.shape[0]), (0, 0)))
    comb = transpose_run(emb0.T, emb1.T, tail)

    run = pl.kernel(
        _make_body(U),
        compiler_params=cp,
        out_type=jax.ShapeDtypeStruct((n_tok, _D), jnp.float32),
        mesh=mesh,
        scratch_types=[
            pltpu.VMEM((per_w,), jnp.int32),
            pltpu.VMEM((2, _CHUNK, _D), jnp.float32),
            pltpu.VMEM((2, _CHUNK, _D), jnp.float32),
            pltpu.VMEM((2, 2 * _L), jnp.float32),
            pltpu.SemaphoreType.DMA,
            pltpu.SemaphoreType.DMA,
            pltpu.SemaphoreType.DMA,
            pltpu.SemaphoreType.DMA,
        ],
    )
    out = run(y_flat, comb).reshape(B, U, _D)
    state = y[:, U - _CONTEXT + 1:]
    return (out, state)


# speedup vs baseline: 1.2950x; 1.0122x over previous
import jax
import jax.numpy as jnp
from jax import lax
from jax.experimental import pallas as pl
from jax.experimental.pallas import tpu as pltpu
from jax.experimental.pallas import tpu_sc as plsc

_CONTEXT = 2
_D0, _D1 = 96, 32
_D = _D0 + _D1
_NC, _NS = 2, 16
_NW = _NC * _NS
_CHUNK = 128
_EPS = 1e-5
_L = 16


def _rsqrt16(x):
    h = x * 0.5
    i = plsc.bitcast(x, jnp.int32)
    g = plsc.bitcast(jnp.full((_L,), 0x5F3759DF, jnp.int32) - (i >> 1),
                     jnp.float32)
    for _ in range(2):
        g = g * (1.5 - h * g * g)
    return g


def _make_body(U):
    def _sc_body(y_hbm, comb_hbm, out_hbm,
                 i_all, gs, outs, tails, g0, g1, w0, w1):
        wid = lax.axis_index("s") * _NC + lax.axis_index("c")
        per_w = out_hbm.shape[0] // _NW
        n_chunks = per_w // _CHUNK
        base_w = wid * per_w
        gsems = (g0, g1)
        wsems = (w0, w1)

        pltpu.sync_copy(y_hbm.at[pl.ds(base_w, per_w)], i_all)

        def gather(slot, ci):
            sl = pl.ds(ci * _CHUNK, _CHUNK)
            return pltpu.make_async_copy(
                comb_hbm.at[i_all.at[sl]], gs.at[slot], gsems[slot])

        def save_tail(slot):
            tails[slot, pl.ds(0, _L)] = gs[slot, _CHUNK - 1, pl.ds(_D0, _L)]
            tails[slot, pl.ds(_L, _L)] = \
                gs[slot, _CHUNK - 1, pl.ds(_D0 + _L, _L)]

        def out_copy(slot, ci):
            base = base_w + ci * _CHUNK
            return pltpu.make_async_copy(
                outs.at[slot], out_hbm.at[pl.ds(base, _CHUNK)], wsems[slot])

        def compute(slot, ci):
            g = gs.at[slot]
            out_v = outs.at[slot]
            base = base_w + ci * _CHUNK

            @plsc.parallel_loop(0, _CHUNK, unroll=16)
            def _tok(t):
                vs = [g[t, pl.ds(_L * j, _L)] for j in range(_D0 // _L)]
                tp = jnp.maximum(t - 1, 0)
                tv = jnp.full((_L,), t, jnp.int32)
                first = tv == 0
                e1a = jnp.where(first, tails[1 - slot, pl.ds(0, _L)],
                                g[tp, pl.ds(_D0, _L)])
                e1b = jnp.where(first, tails[1 - slot, pl.ds(_L, _L)],
                                g[tp, pl.ds(_D0 + _L, _L)])
                rem = lax.rem(base + t, U)
                row0 = jnp.full((_L,), rem, jnp.int32) == 0
                vs.append(jnp.where(row0, 0.0, e1a))
                vs.append(jnp.where(row0, 0.0, e1b))

                s = ((vs[0] + vs[1]) + (vs[2] + vs[3])) + \
                    ((vs[4] + vs[5]) + (vs[6] + vs[7]))
                q = ((vs[0] * vs[0] + vs[1] * vs[1]) +
                     (vs[2] * vs[2] + vs[3] * vs[3])) + \
                    ((vs[4] * vs[4] + vs[5] * vs[5]) +
                     (vs[6] * vs[6] + vs[7] * vs[7]))
                mean = jnp.sum(s) * (1.0 / _D)
                var = jnp.sum(q) * (1.0 / _D) - mean * mean + _EPS
                r = _rsqrt16(jnp.full((_L,), var, jnp.float32))
                m = jnp.full((_L,), mean, jnp.float32)
                for j in range(_D // _L):
                    out_v[t, pl.ds(_L * j, _L)] = (vs[j] - m) * r

        gather(0, 0).start()

        @pl.loop(0, n_chunks, step=2)
        def _pair(c):
            gather(1, c + 1).start()
            gather(0, c).wait()
            save_tail(0)

            @pl.when(c >= 2)
            def _():
                out_copy(0, c).wait()
            compute(0, c)
            out_copy(0, c).start()

            @pl.when(c + 2 < n_chunks)
            def _():
                gather(0, c + 2).start()
            gather(1, c + 1).wait()
            save_tail(1)

            @pl.when(c >= 2)
            def _():
                out_copy(1, c + 1).wait()
            compute(1, c + 1)
            out_copy(1, c + 1).start()

        out_copy(0, n_chunks - 2).wait()
        out_copy(1, n_chunks - 1).wait()

    return _sc_body


_VPAD = 100096
_NFULL = _VPAD // _CHUNK - 1
_TAILV = 100001 - _NFULL * _CHUNK


def _transpose_body(e0t_hbm, e1t_hbm, tail_hbm, comb_hbm,
                    t0s, t1s, obs, i0a, i1a, i0b, i1b, wa, wb):
    wid = lax.axis_index("s") * _NC + lax.axis_index("c")
    lanes = jnp.arange(_L, dtype=jnp.int32)
    isems = ((i0a, i1a), (i0b, i1b))
    wsems = (wa, wb)

    def in_copies(slot, j):
        s0, s1 = isems[slot]
        sl = pl.ds(j * _CHUNK, _CHUNK)
        return (pltpu.make_async_copy(e0t_hbm.at[:, sl], t0s.at[slot], s0),
                pltpu.make_async_copy(e1t_hbm.at[:, sl], t1s.at[slot], s1))

    def out_copy(slot, j):
        return pltpu.make_async_copy(
            obs.at[slot], comb_hbm.at[pl.ds(j * _CHUNK, _CHUNK)],
            wsems[slot])

    def compute(slot):
        t0 = t0s.at[slot]
        t1 = t1s.at[slot]
        ob = obs.at[slot]

        def tile_t(src, dst_col0, i):
            fb = (i >> 3) * _L
            vb = (i & 7) * _L
            fcol = dst_col0 + fb + lanes
            for k in range(_L):
                rot = vb + ((lanes + k) & (_L - 1))
                d = plsc.load_gather(src, [fb + lanes, rot])
                plsc.store_scatter(ob, [rot, fcol], d)

        @plsc.parallel_loop(0, (_D0 // _L) * 8, unroll=2)
        def _t0tile(i):
            tile_t(t0, 0, i)

        @plsc.parallel_loop(0, (_D1 // _L) * 8, unroll=2)
        def _t1tile(i):
            tile_t(t1, _D0, i)

    def step(slot, j, k):
        @pl.when(j < _NFULL)
        def _():
            a, b = in_copies(slot, j)
            a.wait()
            b.wait()

            @pl.when(k >= 2)
            def _():
                out_copy(slot, j).wait()
            compute(slot)
            out_copy(slot, j).start()

    def fire(slot, j):
        @pl.when(j < _NFULL)
        def _():
            a, b = in_copies(slot, j)
            a.start()
            b.start()

    fire(0, wid)

    @pl.loop(0, 25, step=2)
    def _blk(k):
        j0 = k * _NW + wid
        j1 = j0 + _NW
        fire(1, j1)
        step(0, j0, k)
        fire(0, j1 + _NW)
        step(1, j1, k)

    out_copy(0, wid).wait()
    out_copy(1, wid).wait()

    @pl.when(wid == _NW - 1)
    def _():
        nt = tail_hbm.shape[0]
        pltpu.sync_copy(tail_hbm, obs.at[0, pl.ds(0, nt)])
        pltpu.sync_copy(obs.at[0, pl.ds(0, nt)],
                        comb_hbm.at[pl.ds(_NFULL * _CHUNK, nt)])


def kernel(y, emb0, emb1):
    B, U = y.shape
    n_tok = B * U
    per_w = n_tok // _NW
    y_flat = y.reshape(n_tok)

    cp = pltpu.CompilerParams(
        needs_layout_passes=False, use_tc_tiling_on_sc=True)
    mesh = plsc.VectorSubcoreMesh(core_axis_name="c", subcore_axis_name="s")

    transpose_run = pl.kernel(
        _transpose_body,
        compiler_params=cp,
        out_type=jax.ShapeDtypeStruct((_VPAD, _D), jnp.float32),
        mesh=mesh,
        scratch_types=[
            pltpu.VMEM((2, _D0, _CHUNK), jnp.float32),
            pltpu.VMEM((2, _D1, _CHUNK), jnp.float32),
            pltpu.VMEM((2, _CHUNK, _D), jnp.float32),
            pltpu.SemaphoreType.DMA,
            pltpu.SemaphoreType.DMA,
            pltpu.SemaphoreType.DMA,
            pltpu.SemaphoreType.DMA,
            pltpu.SemaphoreType.DMA,
            pltpu.SemaphoreType.DMA,
        ],
    )
    vtail = _NFULL * _CHUNK
    tail = jnp.concatenate([emb0[vtail:], emb1[vtail:]], axis=1)
    tail = jnp.pad(tail, ((0, 40 - tail.shape[0]), (0, 0)))
    comb = transpose_run(emb0.T, emb1.T, tail)

    run = pl.kernel(
        _make_body(U),
        compiler_params=cp,
        out_type=jax.ShapeDtypeStruct((n_tok, _D), jnp.float32),
        mesh=mesh,
        scratch_types=[
            pltpu.VMEM((per_w,), jnp.int32),
            pltpu.VMEM((2, _CHUNK, _D), jnp.float32),
            pltpu.VMEM((2, _CHUNK, _D), jnp.float32),
            pltpu.VMEM((2, 2 * _L), jnp.float32),
            pltpu.SemaphoreType.DMA,
            pltpu.SemaphoreType.DMA,
            pltpu.SemaphoreType.DMA,
            pltpu.SemaphoreType.DMA,
        ],
    )
    out = run(y_flat, comb).reshape(B, U, _D)
    state = y[:, U - _CONTEXT + 1:]
    return (out, state)


# speedup vs baseline: 1.2975x; 1.0020x over previous
import jax
import jax.numpy as jnp
from jax import lax
from jax.experimental import pallas as pl
from jax.experimental.pallas import tpu as pltpu
from jax.experimental.pallas import tpu_sc as plsc

_CONTEXT = 2
_D0, _D1 = 96, 32
_D = _D0 + _D1
_NC, _NS = 2, 16
_NW = _NC * _NS
_CHUNK = 128
_EPS = 1e-5
_L = 16


def _rsqrt16(x):
    h = x * 0.5
    i = plsc.bitcast(x, jnp.int32)
    g = plsc.bitcast(jnp.full((_L,), 0x5F3759DF, jnp.int32) - (i >> 1),
                     jnp.float32)
    for _ in range(2):
        g = g * (1.5 - h * g * g)
    return g


def _make_body(U):
    def _sc_body(y_hbm, comb_hbm, out_hbm,
                 i_all, gs, outs, tails, g0, g1, w0, w1):
        wid = lax.axis_index("s") * _NC + lax.axis_index("c")
        per_w = out_hbm.shape[0] // _NW
        n_chunks = per_w // _CHUNK
        base_w = wid * per_w
        gsems = (g0, g1)
        wsems = (w0, w1)

        pltpu.sync_copy(y_hbm.at[pl.ds(base_w, per_w)], i_all)

        def gather(slot, ci):
            sl = pl.ds(ci * _CHUNK, _CHUNK)
            return pltpu.make_async_copy(
                comb_hbm.at[i_all.at[sl]], gs.at[slot], gsems[slot])

        def save_tail(slot):
            tails[slot, pl.ds(0, _L)] = gs[slot, _CHUNK - 1, pl.ds(_D0, _L)]
            tails[slot, pl.ds(_L, _L)] = \
                gs[slot, _CHUNK - 1, pl.ds(_D0 + _L, _L)]

        def out_copy(slot, ci):
            base = base_w + ci * _CHUNK
            return pltpu.make_async_copy(
                outs.at[slot], out_hbm.at[pl.ds(base, _CHUNK)], wsems[slot])

        def compute(slot, ci):
            g = gs.at[slot]
            out_v = outs.at[slot]
            base = base_w + ci * _CHUNK

            @plsc.parallel_loop(0, _CHUNK, unroll=16)
            def _tok(t):
                vs = [g[t, pl.ds(_L * j, _L)] for j in range(_D0 // _L)]
                tp = jnp.maximum(t - 1, 0)
                tv = jnp.full((_L,), t, jnp.int32)
                first = tv == 0
                e1a = jnp.where(first, tails[1 - slot, pl.ds(0, _L)],
                                g[tp, pl.ds(_D0, _L)])
                e1b = jnp.where(first, tails[1 - slot, pl.ds(_L, _L)],
                                g[tp, pl.ds(_D0 + _L, _L)])
                rem = lax.rem(base + t, U)
                row0 = jnp.full((_L,), rem, jnp.int32) == 0
                vs.append(jnp.where(row0, 0.0, e1a))
                vs.append(jnp.where(row0, 0.0, e1b))

                s = ((vs[0] + vs[1]) + (vs[2] + vs[3])) + \
                    ((vs[4] + vs[5]) + (vs[6] + vs[7]))
                q = ((vs[0] * vs[0] + vs[1] * vs[1]) +
                     (vs[2] * vs[2] + vs[3] * vs[3])) + \
                    ((vs[4] * vs[4] + vs[5] * vs[5]) +
                     (vs[6] * vs[6] + vs[7] * vs[7]))
                mean = jnp.sum(s) * (1.0 / _D)
                var = jnp.sum(q) * (1.0 / _D) - mean * mean + _EPS
                r = _rsqrt16(jnp.full((_L,), var, jnp.float32))
                m = jnp.full((_L,), mean, jnp.float32)
                for j in range(_D // _L):
                    out_v[t, pl.ds(_L * j, _L)] = (vs[j] - m) * r

        gather(0, 0).start()

        @pl.loop(0, n_chunks, step=2)
        def _pair(c):
            gather(1, c + 1).start()
            gather(0, c).wait()
            save_tail(0)

            @pl.when(c >= 2)
            def _():
                out_copy(0, c).wait()
            compute(0, c)
            out_copy(0, c).start()

            @pl.when(c + 2 < n_chunks)
            def _():
                gather(0, c + 2).start()
            gather(1, c + 1).wait()
            save_tail(1)

            @pl.when(c >= 2)
            def _():
                out_copy(1, c + 1).wait()
            compute(1, c + 1)
            out_copy(1, c + 1).start()

        out_copy(0, n_chunks - 2).wait()
        out_copy(1, n_chunks - 1).wait()

    return _sc_body


_VPAD = 100096
_NFULL = _VPAD // _CHUNK - 1


def _transpose_body(e0t_hbm, e1t_hbm, tail_hbm, comb_hbm,
                    t0s, t1s, obs, i0a, i1a, i0b, i1b, wa, wb):
    wid = lax.axis_index("s") * _NC + lax.axis_index("c")
    lanes = jnp.arange(_L, dtype=jnp.int32)
    isems = ((i0a, i1a), (i0b, i1b))
    wsems = (wa, wb)

    def in_copies(slot, j):
        s0, s1 = isems[slot]
        sl = pl.ds(j * _CHUNK, _CHUNK)
        return (pltpu.make_async_copy(e0t_hbm.at[:, sl], t0s.at[slot], s0),
                pltpu.make_async_copy(e1t_hbm.at[:, sl], t1s.at[slot], s1))

    def out_copy(slot, j):
        return pltpu.make_async_copy(
            obs.at[slot], comb_hbm.at[pl.ds(j * _CHUNK, _CHUNK)],
            wsems[slot])

    def compute(slot):
        t0 = t0s.at[slot]
        t1 = t1s.at[slot]
        ob = obs.at[slot]

        def tile_t(src, dst_col0, i):
            fb = (i >> 3) * _L
            vb = (i & 7) * _L
            fcol = dst_col0 + fb + lanes
            for k in range(_L):
                rot = vb + ((lanes + k) & (_L - 1))
                d = plsc.load_gather(src, [fb + lanes, rot])
                plsc.store_scatter(ob, [rot, fcol], d)

        @plsc.parallel_loop(0, (_D0 // _L) * 8, unroll=2)
        def _t0tile(i):
            tile_t(t0, 0, i)

        @plsc.parallel_loop(0, (_D1 // _L) * 8, unroll=2)
        def _t1tile(i):
            tile_t(t1, _D0, i)

    def step(slot, j, k):
        @pl.when(j < _NFULL)
        def _():
            a, b = in_copies(slot, j)
            a.wait()
            b.wait()

            @pl.when(k >= 2)
            def _():
                out_copy(slot, j).wait()
            compute(slot)
            out_copy(slot, j).start()

    def fire(slot, j):
        @pl.when(j < _NFULL)
        def _():
            a, b = in_copies(slot, j)
            a.start()
            b.start()

    fire(0, wid)

    @pl.loop(0, 25, step=2)
    def _blk(k):
        j0 = k * _NW + wid
        j1 = j0 + _NW
        fire(1, j1)
        step(0, j0, k)
        fire(0, j1 + _NW)
        step(1, j1, k)

    out_copy(0, wid).wait()
    out_copy(1, wid).wait()

    @pl.when(wid == _NW - 1)
    def _():
        nt = tail_hbm.shape[0]
        pltpu.sync_copy(tail_hbm, obs.at[0, pl.ds(0, nt)])
        pltpu.sync_copy(obs.at[0, pl.ds(0, nt)],
                        comb_hbm.at[pl.ds(_NFULL * _CHUNK, nt)])


def kernel(y, emb0, emb1):
    B, U = y.shape
    n_tok = B * U
    per_w = n_tok // _NW
    y_flat = y.reshape(n_tok)

    cp = pltpu.CompilerParams(
        needs_layout_passes=False, use_tc_tiling_on_sc=True)
    mesh = plsc.VectorSubcoreMesh(core_axis_name="c", subcore_axis_name="s")

    transpose_run = pl.kernel(
        _transpose_body,
        compiler_params=cp,
        out_type=jax.ShapeDtypeStruct((_VPAD, _D), jnp.float32),
        mesh=mesh,
        scratch_types=[
            pltpu.VMEM((2, _D0, _CHUNK), jnp.float32),
            pltpu.VMEM((2, _D1, _CHUNK), jnp.float32),
            pltpu.VMEM((2, _CHUNK, _D), jnp.float32),
            pltpu.SemaphoreType.DMA,
            pltpu.SemaphoreType.DMA,
            pltpu.SemaphoreType.DMA,
            pltpu.SemaphoreType.DMA,
            pltpu.SemaphoreType.DMA,
            pltpu.SemaphoreType.DMA,
        ],
    )
    vtail = _NFULL * _CHUNK
    tail = jnp.concatenate([emb0[vtail:], emb1[vtail:]], axis=1)
    tail = jnp.pad(tail, ((0, 40 - tail.shape[0]), (0, 0)))
    comb = transpose_run(emb0.T, emb1.T, tail)

    run = pl.kernel(
        _make_body(U),
        compiler_params=cp,
        out_type=jax.ShapeDtypeStruct((n_tok, _D), jnp.float32),
        mesh=mesh,
        scratch_types=[
            pltpu.VMEM((per_w,), jnp.int32),
            pltpu.VMEM((2, _CHUNK, _D), jnp.float32),
            pltpu.VMEM((2, _CHUNK, _D), jnp.float32),
            pltpu.VMEM((2, 2 * _L), jnp.float32),
            pltpu.SemaphoreType.DMA,
            pltpu.SemaphoreType.DMA,
            pltpu.SemaphoreType.DMA,
            pltpu.SemaphoreType.DMA,
        ],
    )
    out = run(y_flat, comb).reshape(B, U, _D)
    state = y[:, U - _CONTEXT + 1:]
    return (out, state)
